# baseline (speedup 1.0000x reference)
import jax
import jax.numpy as jnp
from jax import lax
from jax.experimental import pallas as pl
from jax.experimental.pallas import tpu as pltpu

N = 8192
D = 128
NCHUNK = 8
CH = N // NCHUNK


def _dma_kernel(pos_ref, e_ref, emb_ref, w_hbm, new_emb_ref, out_hbm,
                sc_ref, sr_ref, stripe_ref, band_ref, copy_sem, fix_sem):
    pos = pos_ref[0]
    c0 = pl.multiple_of((pos // 128) * 128, 128)
    r0 = pl.multiple_of((pos // 8) * 8, 8)

    copies = [
        pltpu.make_async_copy(
            w_hbm.at[pl.ds(k * CH, CH), :],
            out_hbm.at[pl.ds(k * CH, CH), :],
            copy_sem,
        )
        for k in range(NCHUNK)
    ]
    for c in copies:
        c.start()

    stripe_load = pltpu.make_async_copy(
        w_hbm.at[:, pl.ds(c0, 128)], stripe_ref, fix_sem)
    band_load = pltpu.make_async_copy(
        w_hbm.at[pl.ds(r0, 8), :], band_ref, fix_sem)
    stripe_load.start()
    band_load.start()

    E = emb_ref[...]
    ev = e_ref[...]
    dots_c = lax.dot_general(E, ev, (((1,), (1,)), ((), ())),
                             preferred_element_type=jnp.float32)
    n2_c = jnp.sum(E * E, axis=1, keepdims=True)
    sims_c = dots_c / (jnp.sqrt(n2_c) + 1e-8)
    sc_ref[...] = sims_c
    dots_r = lax.dot_general(ev, E, (((1,), (1,)), ((), ())),
                             preferred_element_type=jnp.float32)
    ones = jnp.ones((1, D), jnp.float32)
    n2_r = lax.dot_general(ones, E * E, (((1,), (1,)), ((), ())),
                           preferred_element_type=jnp.float32)
    sims_r = dots_r / (jnp.sqrt(n2_r) + 1e-8)
    sr_ref[...] = sims_r
    rows0 = lax.broadcasted_iota(jnp.int32, (N, D), 0)
    new_emb_ref[...] = jnp.where(rows0 == pos, ev, E)

    stripe_load.wait()
    band_load.wait()
    s_rows = lax.broadcasted_iota(jnp.int32, (N, 128), 0)
    s_cols = lax.broadcasted_iota(jnp.int32, (N, 128), 1)
    stripe = stripe_ref[...]
    stripe = jnp.where(s_cols == pos - c0, sc_ref[...], stripe)
    sr_slice = sr_ref[:, pl.ds(c0, 128)]
    stripe = jnp.where(s_rows == pos, sr_slice, stripe)
    stripe_ref[...] = stripe

    b_rows = lax.broadcasted_iota(jnp.int32, (8, N), 0)
    b_cols = lax.broadcasted_iota(jnp.int32, (8, N), 1)
    band = band_ref[...]
    sc_slice = sc_ref[pl.ds(r0, 8), :]
    band = jnp.where(b_cols == pos, sc_slice, band)
    band = jnp.where(b_rows == pos - r0, sr_ref[...], band)
    band_ref[...] = band

    for c in copies:
        c.wait()
    stripe_store = pltpu.make_async_copy(
        stripe_ref, out_hbm.at[:, pl.ds(c0, 128)], fix_sem)
    band_store = pltpu.make_async_copy(
        band_ref, out_hbm.at[pl.ds(r0, 8), :], fix_sem)
    stripe_store.start()
    band_store.start()
    stripe_store.wait()
    band_store.wait()


def kernel(experience_embeddings, associative_weights, experience,
           temporal_context, position):
    del temporal_context
    pos = jnp.asarray(position, jnp.int32).reshape(1)
    e2 = experience.reshape(1, D)

    new_emb, new_w = pl.pallas_call(
        _dma_kernel,
        out_shape=(jax.ShapeDtypeStruct((N, D), jnp.float32),
                   jax.ShapeDtypeStruct((N, N), jnp.float32)),
        in_specs=[pl.BlockSpec(memory_space=pltpu.SMEM),
                  pl.BlockSpec((1, D), lambda: (0, 0)),
                  pl.BlockSpec((N, D), lambda: (0, 0)),
                  pl.BlockSpec(memory_space=pl.ANY)],
        out_specs=(pl.BlockSpec((N, D), lambda: (0, 0)),
                   pl.BlockSpec(memory_space=pl.ANY)),
        scratch_shapes=[pltpu.VMEM((N, 1), jnp.float32),
                        pltpu.VMEM((1, N), jnp.float32),
                        pltpu.VMEM((N, 128), jnp.float32),
                        pltpu.VMEM((8, N), jnp.float32),
                        pltpu.SemaphoreType.DMA,
                        pltpu.SemaphoreType.DMA],
    )(pos, e2, experience_embeddings, associative_weights)

    return (new_emb, new_w)


# speedup vs baseline: 47.0605x; 47.0605x over previous
import jax
import jax.numpy as jnp
from jax import lax
from jax.experimental import pallas as pl
from jax.experimental.pallas import tpu as pltpu

N = 8192
D = 128
BLK = 256


def _fused_kernel(pos_ref, e_ref, embf_ref, emb_ref, w_ref,
                  new_emb_ref, out_ref):
    i = pl.program_id(0)
    pos = pos_ref[0]
    ev = e_ref[...]

    E_blk = emb_ref[...]
    dots_c = lax.dot_general(E_blk, ev, (((1,), (1,)), ((), ())),
                             preferred_element_type=jnp.float32)
    n2_c = jnp.sum(E_blk * E_blk, axis=1, keepdims=True)
    sc_blk = dots_c / (jnp.sqrt(n2_c) + 1e-8)

    rows_d = lax.broadcasted_iota(jnp.int32, (BLK, D), 0) + i * BLK
    new_emb_ref[...] = jnp.where(rows_d == pos, ev, E_blk)

    W = w_ref[...]
    rows = lax.broadcasted_iota(jnp.int32, (BLK, N), 0) + i * BLK
    cols = lax.broadcasted_iota(jnp.int32, (BLK, N), 1)
    W = jnp.where(cols == pos, sc_blk, W)
    out_ref[...] = W

    @pl.when((pos >= i * BLK) & (pos < (i + 1) * BLK))
    def _():
        E = embf_ref[...]
        dots_r = lax.dot_general(ev, E, (((1,), (1,)), ((), ())),
                                 preferred_element_type=jnp.float32)
        ones = jnp.ones((1, D), jnp.float32)
        n2_r = lax.dot_general(ones, E * E, (((1,), (1,)), ((), ())),
                               preferred_element_type=jnp.float32)
        sr = dots_r / (jnp.sqrt(n2_r) + 1e-8)
        out_ref[pl.ds(pos - i * BLK, 1), :] = sr


def kernel(experience_embeddings, associative_weights, experience,
           temporal_context, position):
    del temporal_context
    pos = jnp.asarray(position, jnp.int32).reshape(1)
    e2 = experience.reshape(1, D)

    new_emb, new_w = pl.pallas_call(
        _fused_kernel,
        grid=(N // BLK,),
        out_shape=(jax.ShapeDtypeStruct((N, D), jnp.float32),
                   jax.ShapeDtypeStruct((N, N), jnp.float32)),
        in_specs=[pl.BlockSpec(memory_space=pltpu.SMEM),
                  pl.BlockSpec((1, D), lambda i: (0, 0)),
                  pl.BlockSpec((N, D), lambda i: (0, 0)),
                  pl.BlockSpec((BLK, D), lambda i: (i, 0)),
                  pl.BlockSpec((BLK, N), lambda i: (i, 0))],
        out_specs=(pl.BlockSpec((BLK, D), lambda i: (i, 0)),
                   pl.BlockSpec((BLK, N), lambda i: (i, 0))),
        compiler_params=pltpu.CompilerParams(
            dimension_semantics=("parallel",)),
    )(pos, e2, experience_embeddings, experience_embeddings,
      associative_weights)

    return (new_emb, new_w)
